# Initial kernel scaffold; baseline (speedup 1.0000x reference)
import jax
import jax.numpy as jnp
from jax.experimental import pallas as pl


def kernel(images, gt_boxes, gt_classes, anchor_boxes):
    raise NotImplementedError("write your pallas kernel here")


# speedup vs baseline: 2.9938x; 2.9938x over previous
import functools

import jax
import jax.numpy as jnp
from jax.experimental import pallas as pl

_T = 512
_NPAD = 128


def _encode_kernel(a_ref, rows_ref, box_ref, cls_ref):
    a = a_ref[...]
    ax1 = a[:, 0:1]
    ay1 = a[:, 1:2]
    ax2 = a[:, 2:3]
    ay2 = a[:, 3:4]

    rows = rows_ref[0]
    bx1 = rows[0:1, :]
    by1 = rows[1:2, :]
    bx2 = rows[2:3, :]
    by2 = rows[3:4, :]
    barea = rows[4:5, :]
    bcls = rows[5:6, :]

    iw = jnp.maximum(jnp.minimum(ax2, bx2) - jnp.maximum(ax1, bx1), 0.0)
    ih = jnp.maximum(jnp.minimum(ay2, by2) - jnp.maximum(ay1, by1), 0.0)
    inter = iw * ih
    area_a = (ax2 - ax1) * (ay2 - ay1)
    union = area_a + barea - inter
    iou = inter / jnp.maximum(union, 1e-8)

    max_iou = jnp.max(iou, axis=1, keepdims=True)
    lane = jax.lax.broadcasted_iota(jnp.int32, iou.shape, 1)
    midx = jnp.min(jnp.where(iou == max_iou, lane, _NPAD), axis=1,
                   keepdims=True)
    onehot = (lane == midx).astype(jnp.float32)

    def sel(row):
        return jnp.sum(onehot * row, axis=1, keepdims=True)

    gx1 = sel(bx1)
    gy1 = sel(by1)
    gx2 = sel(bx2)
    gy2 = sel(by2)
    gcls = sel(bcls)

    aw = ax2 - ax1
    ah = ay2 - ay1
    acx = ax1 + aw * 0.5
    acy = ay1 + ah * 0.5
    gw = gx2 - gx1
    gh = gy2 - gy1
    gcx = gx1 + gw * 0.5
    gcy = gy1 + gh * 0.5

    tx = ((gcx - acx) / aw) / 0.1
    ty = ((gcy - acy) / ah) / 0.1
    tw = jnp.log(gw / aw) / 0.2
    th = jnp.log(gh / ah) / 0.2
    bt = jnp.concatenate([tx, ty, tw, th], axis=1)
    bt = jnp.where(jnp.isnan(bt), -2.0, bt)

    pos = max_iou >= 0.5
    ign = jnp.logical_and(max_iou >= 0.4, max_iou < 0.5)
    cls = jnp.where(pos, gcls, -1.0)
    cls = jnp.where(ign, -2.0, cls)
    cls = jnp.where(jnp.isnan(cls), -2.0, cls)

    box_ref[0] = bt
    cls_ref[0] = cls


@functools.partial(jax.jit, static_argnums=())
def kernel(images, gt_boxes, gt_classes, anchor_boxes):
    del images
    B, N = gt_classes.shape
    A = anchor_boxes.shape[0]
    G = -(-A // _T)
    A_pad = G * _T

    pad_box = jnp.broadcast_to(
        jnp.asarray([0.0, 0.0, 1.0, 1.0], jnp.float32), (A_pad - A, 4))
    anchors_p = jnp.concatenate([anchor_boxes, pad_box], axis=0)

    x1, y1, x2, y2 = (gt_boxes[..., i] for i in range(4))
    area = (x2 - x1) * (y2 - y1)
    zeros = jnp.zeros_like(x1)
    rows = jnp.stack([x1, y1, x2, y2, area, gt_classes, zeros, zeros],
                     axis=1)
    gt_rows = jnp.pad(rows, ((0, 0), (0, 0), (0, _NPAD - N)))

    box_p, cls_p = pl.pallas_call(
        _encode_kernel,
        grid=(G, B),
        in_specs=[
            pl.BlockSpec((_T, 4), lambda g, b: (g, 0)),
            pl.BlockSpec((1, 8, _NPAD), lambda g, b: (b, 0, 0)),
        ],
        out_specs=[
            pl.BlockSpec((1, _T, 4), lambda g, b: (b, g, 0)),
            pl.BlockSpec((1, _T, 1), lambda g, b: (b, g, 0)),
        ],
        out_shape=[
            jax.ShapeDtypeStruct((B, A_pad, 4), jnp.float32),
            jax.ShapeDtypeStruct((B, A_pad, 1), jnp.float32),
        ],
    )(anchors_p, gt_rows)

    return box_p[:, :A, :], cls_p[:, :A, 0]


# speedup vs baseline: 8.0663x; 2.6943x over previous
import functools

import jax
import jax.numpy as jnp
from jax.experimental import pallas as pl

_L = 512
_NPAD = 104


def _encode_kernel(a_ref, g_ref, o_ref):
    a = a_ref[...]
    ax1 = a[0:1, :]
    ay1 = a[1:2, :]
    ax2 = a[2:3, :]
    ay2 = a[3:4, :]
    aw = a[4:5, :]
    ah = a[5:6, :]
    acx = a[6:7, :]
    acy = a[7:8, :]

    g = g_ref[0]
    bx1 = g[:, 0:1]
    by1 = g[:, 1:2]
    bx2 = g[:, 2:3]
    by2 = g[:, 3:4]
    barea = g[:, 4:5]
    bcls = g[:, 5:6]

    iw = jnp.maximum(jnp.minimum(ax2, bx2) - jnp.maximum(ax1, bx1), 0.0)
    ih = jnp.maximum(jnp.minimum(ay2, by2) - jnp.maximum(ay1, by1), 0.0)
    inter = iw * ih
    area_a = aw * ah
    union = area_a + barea - inter
    iou = inter / jnp.maximum(union, 1e-8)

    max_iou = jnp.max(iou, axis=0, keepdims=True)
    sub = jax.lax.broadcasted_iota(jnp.int32, iou.shape, 0)
    midx = jnp.min(jnp.where(iou == max_iou, sub, _NPAD), axis=0,
                   keepdims=True)
    onehot = (sub == midx).astype(jnp.float32)

    def sel(col):
        return jnp.sum(onehot * col, axis=0, keepdims=True)

    gx1 = sel(bx1)
    gy1 = sel(by1)
    gx2 = sel(bx2)
    gy2 = sel(by2)
    gcls = sel(bcls)

    gw = gx2 - gx1
    gh = gy2 - gy1
    gcx = gx1 + gw * 0.5
    gcy = gy1 + gh * 0.5

    tx = ((gcx - acx) / aw) / 0.1
    ty = ((gcy - acy) / ah) / 0.1
    tw = jnp.log(gw / aw) / 0.2
    th = jnp.log(gh / ah) / 0.2

    pos = max_iou >= 0.5
    ign = jnp.logical_and(max_iou >= 0.4, max_iou < 0.5)
    cls = jnp.where(pos, gcls, -1.0)
    cls = jnp.where(ign, -2.0, cls)

    out = jnp.concatenate(
        [tx, ty, tw, th, cls, cls, cls, cls], axis=0)
    out = jnp.where(jnp.isnan(out), -2.0, out)
    o_ref[0] = out


@functools.partial(jax.jit, static_argnums=())
def kernel(images, gt_boxes, gt_classes, anchor_boxes):
    del images
    B, N = gt_classes.shape
    A = anchor_boxes.shape[0]
    G = -(-A // _L)
    A_pad = G * _L

    x1, y1, x2, y2 = (anchor_boxes[:, i] for i in range(4))
    aw = x2 - x1
    ah = y2 - y1
    acx = x1 + aw * 0.5
    acy = y1 + ah * 0.5
    aT = jnp.stack([x1, y1, x2, y2, aw, ah, acx, acy], axis=0)
    pad = jnp.broadcast_to(
        jnp.asarray([0.0, 0.0, 1.0, 1.0, 1.0, 1.0, 0.5, 0.5],
                    jnp.float32)[:, None], (8, A_pad - A))
    aT = jnp.concatenate([aT, pad], axis=1)

    gx1, gy1, gx2, gy2 = (gt_boxes[..., i] for i in range(4))
    area = (gx2 - gx1) * (gy2 - gy1)
    zeros = jnp.zeros_like(gx1)
    cols = jnp.stack([gx1, gy1, gx2, gy2, area, gt_classes, zeros, zeros],
                     axis=-1)
    gt_cols = jnp.pad(cols, ((0, 0), (0, _NPAD - N), (0, 0)))

    out = pl.pallas_call(
        _encode_kernel,
        grid=(G, B),
        in_specs=[
            pl.BlockSpec((8, _L), lambda g, b: (0, g)),
            pl.BlockSpec((1, _NPAD, 8), lambda g, b: (b, 0, 0)),
        ],
        out_specs=pl.BlockSpec((1, 8, _L), lambda g, b: (b, 0, g)),
        out_shape=jax.ShapeDtypeStruct((B, 8, A_pad), jnp.float32),
    )(aT, gt_cols)

    box = jnp.transpose(out[:, 0:4, :A], (0, 2, 1))
    cls = out[:, 4, :A]
    return box, cls


# speedup vs baseline: 12.6591x; 1.5694x over previous
import functools

import jax
import jax.numpy as jnp
from jax.experimental import pallas as pl

_L = 1024
_NPAD = 104


def _encode_kernel(a_ref, g_ref, o_ref):
    a = a_ref[...]
    ax1 = a[0:1, :]
    ay1 = a[1:2, :]
    ax2 = a[2:3, :]
    ay2 = a[3:4, :]
    aw = a[4:5, :]
    ah = a[5:6, :]
    acx = a[6:7, :]
    acy = a[7:8, :]

    g = g_ref[0]
    bx1 = g[:, 0:1]
    by1 = g[:, 1:2]
    bx2 = g[:, 2:3]
    by2 = g[:, 3:4]
    barea = g[:, 4:5]
    bcls = g[:, 5:6]

    iw = jnp.maximum(jnp.minimum(ax2, bx2) - jnp.maximum(ax1, bx1), 0.0)
    ih = jnp.maximum(jnp.minimum(ay2, by2) - jnp.maximum(ay1, by1), 0.0)
    inter = iw * ih
    area_a = aw * ah
    union = area_a + barea - inter
    iou = inter / jnp.maximum(union, 1e-8)

    max_iou = jnp.max(iou, axis=0, keepdims=True)
    sub = jax.lax.broadcasted_iota(jnp.int32, iou.shape, 0)
    midx = jnp.min(jnp.where(iou == max_iou, sub, _NPAD), axis=0,
                   keepdims=True)
    onehot = (sub == midx).astype(jnp.float32)

    def sel(col):
        return jnp.sum(onehot * col, axis=0, keepdims=True)

    gx1 = sel(bx1)
    gy1 = sel(by1)
    gx2 = sel(bx2)
    gy2 = sel(by2)
    gcls = sel(bcls)

    gw = gx2 - gx1
    gh = gy2 - gy1
    gcx = gx1 + gw * 0.5
    gcy = gy1 + gh * 0.5

    tx = ((gcx - acx) / aw) / 0.1
    ty = ((gcy - acy) / ah) / 0.1
    tw = jnp.log(gw / aw) / 0.2
    th = jnp.log(gh / ah) / 0.2

    pos = max_iou >= 0.5
    ign = jnp.logical_and(max_iou >= 0.4, max_iou < 0.5)
    cls = jnp.where(pos, gcls, -1.0)
    cls = jnp.where(ign, -2.0, cls)

    out = jnp.concatenate(
        [tx, ty, tw, th, cls, cls, cls, cls], axis=0)
    out = jnp.where(jnp.isnan(out), -2.0, out)
    o_ref[0] = out


@functools.partial(jax.jit, static_argnums=())
def kernel(images, gt_boxes, gt_classes, anchor_boxes):
    del images
    B, N = gt_classes.shape
    A = anchor_boxes.shape[0]
    G = -(-A // _L)
    A_pad = G * _L

    x1, y1, x2, y2 = (anchor_boxes[:, i] for i in range(4))
    aw = x2 - x1
    ah = y2 - y1
    acx = x1 + aw * 0.5
    acy = y1 + ah * 0.5
    aT = jnp.stack([x1, y1, x2, y2, aw, ah, acx, acy], axis=0)
    pad = jnp.broadcast_to(
        jnp.asarray([0.0, 0.0, 1.0, 1.0, 1.0, 1.0, 0.5, 0.5],
                    jnp.float32)[:, None], (8, A_pad - A))
    aT = jnp.concatenate([aT, pad], axis=1)

    gx1, gy1, gx2, gy2 = (gt_boxes[..., i] for i in range(4))
    area = (gx2 - gx1) * (gy2 - gy1)
    zeros = jnp.zeros_like(gx1)
    cols = jnp.stack([gx1, gy1, gx2, gy2, area, gt_classes, zeros, zeros],
                     axis=-1)
    gt_cols = jnp.pad(cols, ((0, 0), (0, _NPAD - N), (0, 0)))

    out = pl.pallas_call(
        _encode_kernel,
        grid=(G, B),
        in_specs=[
            pl.BlockSpec((8, _L), lambda g, b: (0, g)),
            pl.BlockSpec((1, _NPAD, 8), lambda g, b: (b, 0, 0)),
        ],
        out_specs=pl.BlockSpec((1, 8, _L), lambda g, b: (b, 0, g)),
        out_shape=jax.ShapeDtypeStruct((B, 8, A_pad), jnp.float32),
    )(aT, gt_cols)

    box = jnp.transpose(out[:, 0:4, :A], (0, 2, 1))
    cls = out[:, 4, :A]
    return box, cls


# speedup vs baseline: 14.7780x; 1.1674x over previous
import functools

import jax
import jax.numpy as jnp
from jax.experimental import pallas as pl

_L = 2048
_NPAD = 104


def _encode_kernel(a_ref, g_ref, o_ref):
    a = a_ref[...]
    ax1 = a[0:1, :]
    ay1 = a[1:2, :]
    ax2 = a[2:3, :]
    ay2 = a[3:4, :]
    aw = a[4:5, :]
    ah = a[5:6, :]
    acx = a[6:7, :]
    acy = a[7:8, :]

    g = g_ref[0]
    bx1 = g[:, 0:1]
    by1 = g[:, 1:2]
    bx2 = g[:, 2:3]
    by2 = g[:, 3:4]
    barea = g[:, 4:5]
    bcls = g[:, 5:6]

    iw = jnp.maximum(jnp.minimum(ax2, bx2) - jnp.maximum(ax1, bx1), 0.0)
    ih = jnp.maximum(jnp.minimum(ay2, by2) - jnp.maximum(ay1, by1), 0.0)
    inter = iw * ih
    area_a = aw * ah
    union = area_a + barea - inter
    iou = inter / jnp.maximum(union, 1e-8)

    max_iou = jnp.max(iou, axis=0, keepdims=True)
    sub = jax.lax.broadcasted_iota(jnp.int32, iou.shape, 0)
    midx = jnp.min(jnp.where(iou == max_iou, sub, _NPAD), axis=0,
                   keepdims=True)
    onehot = (sub == midx).astype(jnp.float32)

    def sel(col):
        return jnp.sum(onehot * col, axis=0, keepdims=True)

    gx1 = sel(bx1)
    gy1 = sel(by1)
    gx2 = sel(bx2)
    gy2 = sel(by2)
    gcls = sel(bcls)

    gw = gx2 - gx1
    gh = gy2 - gy1
    gcx = gx1 + gw * 0.5
    gcy = gy1 + gh * 0.5

    tx = ((gcx - acx) / aw) / 0.1
    ty = ((gcy - acy) / ah) / 0.1
    tw = jnp.log(gw / aw) / 0.2
    th = jnp.log(gh / ah) / 0.2

    pos = max_iou >= 0.5
    ign = jnp.logical_and(max_iou >= 0.4, max_iou < 0.5)
    cls = jnp.where(pos, gcls, -1.0)
    cls = jnp.where(ign, -2.0, cls)

    out = jnp.concatenate(
        [tx, ty, tw, th, cls, cls, cls, cls], axis=0)
    out = jnp.where(jnp.isnan(out), -2.0, out)
    o_ref[0] = out


@functools.partial(jax.jit, static_argnums=())
def kernel(images, gt_boxes, gt_classes, anchor_boxes):
    del images
    B, N = gt_classes.shape
    A = anchor_boxes.shape[0]
    G = -(-A // _L)
    A_pad = G * _L

    x1, y1, x2, y2 = (anchor_boxes[:, i] for i in range(4))
    aw = x2 - x1
    ah = y2 - y1
    acx = x1 + aw * 0.5
    acy = y1 + ah * 0.5
    aT = jnp.stack([x1, y1, x2, y2, aw, ah, acx, acy], axis=0)
    pad = jnp.broadcast_to(
        jnp.asarray([0.0, 0.0, 1.0, 1.0, 1.0, 1.0, 0.5, 0.5],
                    jnp.float32)[:, None], (8, A_pad - A))
    aT = jnp.concatenate([aT, pad], axis=1)

    gx1, gy1, gx2, gy2 = (gt_boxes[..., i] for i in range(4))
    area = (gx2 - gx1) * (gy2 - gy1)
    zeros = jnp.zeros_like(gx1)
    cols = jnp.stack([gx1, gy1, gx2, gy2, area, gt_classes, zeros, zeros],
                     axis=-1)
    gt_cols = jnp.pad(cols, ((0, 0), (0, _NPAD - N), (0, 0)))

    out = pl.pallas_call(
        _encode_kernel,
        grid=(G, B),
        in_specs=[
            pl.BlockSpec((8, _L), lambda g, b: (0, g)),
            pl.BlockSpec((1, _NPAD, 8), lambda g, b: (b, 0, 0)),
        ],
        out_specs=pl.BlockSpec((1, 8, _L), lambda g, b: (b, 0, g)),
        out_shape=jax.ShapeDtypeStruct((B, 8, A_pad), jnp.float32),
    )(aT, gt_cols)

    box = jnp.transpose(out[:, 0:4, :A], (0, 2, 1))
    cls = out[:, 4, :A]
    return box, cls
